# baseline (speedup 1.0000x reference)
import functools

import jax
import jax.numpy as jnp
from jax.experimental import pallas as pl
from jax.experimental.pallas import tpu as pltpu

_TOPK = 128
_ROWS_PER_BLOCK = 1024


def _topk_mask_kernel(x_ref, o_ref, keyt_ref, kht_ref, *, k):
    x = x_ref[...]
    rows = x.shape[0]
    xt = x.T
    bt = jax.lax.bitcast_convert_type(xt, jnp.int32)
    keyt = bt ^ jnp.where(bt < 0, jnp.int32(0x7FFFFFFF), jnp.int32(0))
    keyt_ref[...] = keyt
    kht_ref[...] = (keyt >> 16).astype(jnp.int16)

    loa0 = jnp.full((1, rows), -32768, jnp.int32)
    hia0 = jnp.full((1, rows), 32767, jnp.int32)

    def body_a(_, carry):
        lo, hi = carry
        mid = (lo + hi) >> 1
        cnt16 = jnp.sum(
            (kht_ref[...] >= mid.astype(jnp.int16)).astype(jnp.int16),
            axis=0,
            keepdims=True,
        )
        ge = cnt16.astype(jnp.int32) >= k
        return jnp.where(ge, mid, lo), jnp.where(ge, hi, mid)

    loa, _ = jax.lax.fori_loop(0, 16, body_a, (loa0, hia0))

    lob0 = loa << 16
    hib0 = lob0 + jnp.int32(65536)

    def body_b(_, carry):
        lo, hi = carry
        mid = (lo >> 1) + (hi >> 1) + (lo & hi & jnp.int32(1))
        cnt = jnp.sum(
            (keyt_ref[...] >= mid).astype(jnp.int32), axis=0, keepdims=True
        )
        ge = cnt >= k
        return jnp.where(ge, mid, lo), jnp.where(ge, hi, mid)

    lo, _ = jax.lax.fori_loop(0, 16, body_b, (lob0, hib0))
    lo_col = lo.T
    b = jax.lax.bitcast_convert_type(x, jnp.int32)
    key = b ^ jnp.where(b < 0, jnp.int32(0x7FFFFFFF), jnp.int32(0))
    o_ref[...] = jnp.where(key >= lo_col, x, jnp.float32(0))


def kernel(x):
    n, c, h, w = x.shape
    rows = n * c
    hw = h * w
    xr = x.reshape(rows, hw)
    out = pl.pallas_call(
        functools.partial(_topk_mask_kernel, k=_TOPK),
        grid=(rows // _ROWS_PER_BLOCK,),
        in_specs=[pl.BlockSpec((_ROWS_PER_BLOCK, hw), lambda i: (i, 0))],
        out_specs=pl.BlockSpec((_ROWS_PER_BLOCK, hw), lambda i: (i, 0)),
        out_shape=jax.ShapeDtypeStruct((rows, hw), x.dtype),
        scratch_shapes=[
            pltpu.VMEM((hw, _ROWS_PER_BLOCK), jnp.int32),
            pltpu.VMEM((hw, _ROWS_PER_BLOCK), jnp.int16),
        ],
    )(xr)
    return out.reshape(n, c, h, w)


# speedup vs baseline: 1.1888x; 1.1888x over previous
import functools

import jax
import jax.numpy as jnp
from jax.experimental import pallas as pl
from jax.experimental.pallas import tpu as pltpu

_TOPK = 128
_ROWS_PER_BLOCK = 1024


def _topk_mask_kernel(x_ref, o_ref, keyt_ref, *, k):
    x = x_ref[...]
    rows = x.shape[0]
    xt = x.T
    bt = jax.lax.bitcast_convert_type(xt, jnp.int32)
    keyt_ref[...] = bt ^ jnp.where(bt < 0, jnp.int32(0x7FFFFFFF), jnp.int32(0))
    lo0 = jnp.full((1, rows), jnp.iinfo(jnp.int32).min, jnp.int32)
    hi0 = jnp.full((1, rows), jnp.iinfo(jnp.int32).max, jnp.int32)

    def body(_, carry):
        lo, hi = carry
        mid = (lo >> 1) + (hi >> 1) + (lo & hi & jnp.int32(1))
        cnt = jnp.sum(
            (keyt_ref[...] >= mid).astype(jnp.int32), axis=0, keepdims=True
        )
        ge = cnt >= k
        return jnp.where(ge, mid, lo), jnp.where(ge, hi, mid)

    lo, _ = jax.lax.fori_loop(0, 32, body, (lo0, hi0))
    lo_col = lo.T
    b = jax.lax.bitcast_convert_type(x, jnp.int32)
    key = b ^ jnp.where(b < 0, jnp.int32(0x7FFFFFFF), jnp.int32(0))
    o_ref[...] = jnp.where(key >= lo_col, x, jnp.float32(0))


def kernel(x):
    n, c, h, w = x.shape
    rows = n * c
    hw = h * w
    xr = x.reshape(rows, hw)
    out = pl.pallas_call(
        functools.partial(_topk_mask_kernel, k=_TOPK),
        grid=(rows // _ROWS_PER_BLOCK,),
        in_specs=[pl.BlockSpec((_ROWS_PER_BLOCK, hw), lambda i: (i, 0))],
        out_specs=pl.BlockSpec((_ROWS_PER_BLOCK, hw), lambda i: (i, 0)),
        out_shape=jax.ShapeDtypeStruct((rows, hw), x.dtype),
        scratch_shapes=[pltpu.VMEM((hw, _ROWS_PER_BLOCK), jnp.int32)],
    )(xr)
    return out.reshape(n, c, h, w)


# speedup vs baseline: 1.1918x; 1.0025x over previous
import functools

import jax
import jax.numpy as jnp
from jax.experimental import pallas as pl
from jax.experimental.pallas import tpu as pltpu

_TOPK = 128
_ROWS_PER_BLOCK = 1024


def _topk_mask_kernel(x_ref, o_ref, keyt_ref, *, k):
    x = x_ref[...]
    rows = x.shape[0]
    xt = x.T
    bt = jax.lax.bitcast_convert_type(xt, jnp.int32)
    keyt_ref[...] = bt ^ jnp.where(bt < 0, jnp.int32(0x7FFFFFFF), jnp.int32(0))
    lo0 = jnp.full((1, rows), jnp.iinfo(jnp.int32).min, jnp.int32)
    hi0 = jnp.full((1, rows), jnp.iinfo(jnp.int32).max, jnp.int32)

    def body(_, carry):
        lo, hi = carry
        mid = (lo >> 1) + (hi >> 1) + (lo & hi & jnp.int32(1))
        cnt = jnp.sum(
            (keyt_ref[...] >= mid).astype(jnp.int32), axis=0, keepdims=True
        )
        ge = cnt >= k
        return jnp.where(ge, mid, lo), jnp.where(ge, hi, mid)

    lo, _ = jax.lax.fori_loop(0, 32, body, (lo0, hi0), unroll=4)
    lo_col = lo.T
    b = jax.lax.bitcast_convert_type(x, jnp.int32)
    key = b ^ jnp.where(b < 0, jnp.int32(0x7FFFFFFF), jnp.int32(0))
    o_ref[...] = jnp.where(key >= lo_col, x, jnp.float32(0))


def kernel(x):
    n, c, h, w = x.shape
    rows = n * c
    hw = h * w
    xr = x.reshape(rows, hw)
    out = pl.pallas_call(
        functools.partial(_topk_mask_kernel, k=_TOPK),
        grid=(rows // _ROWS_PER_BLOCK,),
        in_specs=[pl.BlockSpec((_ROWS_PER_BLOCK, hw), lambda i: (i, 0))],
        out_specs=pl.BlockSpec((_ROWS_PER_BLOCK, hw), lambda i: (i, 0)),
        out_shape=jax.ShapeDtypeStruct((rows, hw), x.dtype),
        scratch_shapes=[pltpu.VMEM((hw, _ROWS_PER_BLOCK), jnp.int32)],
    )(xr)
    return out.reshape(n, c, h, w)


# speedup vs baseline: 1.5475x; 1.2984x over previous
import functools

import jax
import jax.numpy as jnp
from jax.experimental import pallas as pl
from jax.experimental.pallas import tpu as pltpu

_TOPK = 128
_ROWS_PER_BLOCK = 1024


def _topk_mask_kernel(xt_ref, o_ref, keyt_ref, *, k):
    xt = xt_ref[...]
    rows = xt.shape[1]
    bt = jax.lax.bitcast_convert_type(xt, jnp.int32)
    keyt_ref[...] = bt ^ jnp.where(bt < 0, jnp.int32(0x7FFFFFFF), jnp.int32(0))
    lo0 = jnp.full((1, rows), jnp.iinfo(jnp.int32).min, jnp.int32)
    hi0 = jnp.full((1, rows), jnp.iinfo(jnp.int32).max, jnp.int32)

    def body(_, carry):
        lo, hi = carry
        mid = (lo >> 1) + (hi >> 1) + (lo & hi & jnp.int32(1))
        cnt = jnp.sum(
            (keyt_ref[...] >= mid).astype(jnp.int32), axis=0, keepdims=True
        )
        ge = cnt >= k
        return jnp.where(ge, mid, lo), jnp.where(ge, hi, mid)

    lo, _ = jax.lax.fori_loop(0, 32, body, (lo0, hi0), unroll=4)
    o_ref[...] = jnp.where(keyt_ref[...] >= lo, xt, jnp.float32(0))


def kernel(x):
    n, c, h, w = x.shape
    rows = n * c
    hw = h * w
    xt = x.reshape(rows, hw).T
    out = pl.pallas_call(
        functools.partial(_topk_mask_kernel, k=_TOPK),
        grid=(rows // _ROWS_PER_BLOCK,),
        in_specs=[pl.BlockSpec((hw, _ROWS_PER_BLOCK), lambda i: (0, i))],
        out_specs=pl.BlockSpec((hw, _ROWS_PER_BLOCK), lambda i: (0, i)),
        out_shape=jax.ShapeDtypeStruct((hw, rows), x.dtype),
        scratch_shapes=[pltpu.VMEM((hw, _ROWS_PER_BLOCK), jnp.int32)],
    )(xt)
    return out.T.reshape(n, c, h, w)
